# baseline (speedup 1.0000x reference)
import functools

import jax
import jax.numpy as jnp
from jax import lax
from jax.experimental import pallas as pl
from jax.experimental.pallas import tpu as pltpu
from jax.experimental.pallas import tpu_sc as plsc

D = 64
HW = 256
K = 3
BB = 128
OUT_PAD = 129
NW = 32


def _sc_bow_embed(idx, table, batch):
    chunks = batch // BB
    n_units = HW * chunks // NW
    hw_per_w = n_units // chunks

    mesh = plsc.VectorSubcoreMesh(core_axis_name="c", subcore_axis_name="s")

    @functools.partial(
        pl.kernel,
        out_type=jax.ShapeDtypeStruct((D, 16, 2, batch // 128, 8, 128),
                                      jnp.float32),
        mesh=mesh,
        compiler_params=pltpu.CompilerParams(
            needs_layout_passes=False, use_tc_tiling_on_sc=False),
        scratch_types=[
            pltpu.VMEM((4, K, BB), jnp.int32),
            pltpu.VMEM((4, K * BB, D), jnp.bfloat16),
            pltpu.VMEM((D, OUT_PAD), jnp.float32),
            [pltpu.SemaphoreType.DMA] * 4,
            [pltpu.SemaphoreType.DMA] * 4,
            pltpu.SemaphoreType.DMA,
        ],
    )
    def body(idx_hbm, table_hbm, out_hbm, idx_v, rows_v, out_t,
             gsem, isem, osem):
        wid = lax.axis_index("s") * 2 + lax.axis_index("c")
        lane = lax.iota(jnp.int32, 16)
        third = jnp.float32(1.0 / 3.0)
        e_rows = [c * 16 + lane for c in range(4)]

        def unit_hwb(u):
            hw = wid * hw_per_w + (u // chunks)
            cb = u % chunks
            return hw // 16, hw % 16, cb

        def idx_dma(u, buf):
            h, w, cb = unit_hwb(u)
            return pltpu.make_async_copy(
                idx_hbm.at[:, h, w, cb], idx_v.at[buf], isem[buf])

        def fire(buf):
            for k in range(K):
                pltpu.async_copy(
                    table_hbm.at[idx_v.at[buf, k]],
                    rows_v.at[buf, pl.ds(k * BB, BB)],
                    gsem[buf],
                )

        def drain(buf):
            for k in range(K):
                pltpu.make_async_copy(
                    table_hbm.at[idx_v.at[buf, k]],
                    rows_v.at[buf, pl.ds(k * BB, BB)],
                    gsem[buf],
                ).wait()

        def out_dma(u):
            h, w, cb = unit_hwb(u)
            wt, w8 = w // 8, w % 8
            return pltpu.make_async_copy(
                out_t.at[:, pl.ds(0, BB)],
                out_hbm.at[:, h, wt, cb, w8],
                osem,
            )

        def compute(buf):
            rv = rows_v.at[buf]

            @plsc.parallel_loop(0, BB, 1, unroll=4)
            def _(p):
                col = jnp.zeros((16,), jnp.int32) + p
                for half in range(2):
                    sl = pl.ds(half * 32, 32)
                    parts = [
                        plsc.unpack(rv[k * BB + p, sl],
                                    format=plsc.PackFormat.INTERLEAVED)
                        for k in range(K)
                    ]
                    va = (parts[0][0] + parts[1][0] + parts[2][0]) * third
                    vb = (parts[0][1] + parts[1][1] + parts[2][1]) * third
                    plsc.store_scatter(out_t, [e_rows[2 * half], col], va)
                    plsc.store_scatter(out_t, [e_rows[2 * half + 1], col], vb)

        idx_dma(0, 0).start()
        idx_dma(0, 0).wait()
        fire(0)
        idx_dma(1, 1).start()
        idx_dma(1, 1).wait()
        fire(1)
        idx_dma(2, 2).start()

        def quad_body(qr, _):
            base = qr * 4
            for par in (0, 1, 2, 3):
                u = base + par

                @pl.when(u + 2 < n_units)
                def _():
                    idx_dma(u + 2, (par + 2) & 3).wait()
                    fire((par + 2) & 3)

                @pl.when(u + 3 < n_units)
                def _():
                    idx_dma(u + 3, (par + 3) & 3).start()

                drain(par)

                @pl.when(u > 0)
                def _():
                    out_dma(u - 1).wait()

                compute(par)
                out_dma(u).start()
            return 0

        lax.fori_loop(0, n_units // 4, quad_body, 0)
        out_dma(n_units - 1).wait()

    return body(idx, table)


ROWS_PER_CHUNK = 125


def _sc_table_to_bf16(tpad, v):
    rows_per_w = v // NW
    n_chunks = rows_per_w // ROWS_PER_CHUNK

    mesh = plsc.VectorSubcoreMesh(core_axis_name="c", subcore_axis_name="s")

    @functools.partial(
        pl.kernel,
        out_type=jax.ShapeDtypeStruct((v, D), jnp.bfloat16),
        mesh=mesh,
        compiler_params=pltpu.CompilerParams(
            needs_layout_passes=False, use_tc_tiling_on_sc=False),
        scratch_types=[
            pltpu.VMEM((2, ROWS_PER_CHUNK, 128), jnp.float32),
            pltpu.VMEM((2, ROWS_PER_CHUNK, D), jnp.bfloat16),
            pltpu.SemaphoreType.DMA,
            pltpu.SemaphoreType.DMA,
        ],
    )
    def body(tpad_hbm, out_hbm, inb, outb, insem, outsem):
        wid = lax.axis_index("s") * 2 + lax.axis_index("c")
        r0 = wid * rows_per_w

        def in_dma(c, buf):
            return pltpu.make_async_copy(
                tpad_hbm.at[pl.ds(r0 + c * ROWS_PER_CHUNK, ROWS_PER_CHUNK)],
                inb.at[buf], insem)

        def out_dma(c, buf):
            return pltpu.make_async_copy(
                outb.at[buf],
                out_hbm.at[pl.ds(r0 + c * ROWS_PER_CHUNK, ROWS_PER_CHUNK)],
                outsem)

        def convert(buf):
            @plsc.parallel_loop(0, ROWS_PER_CHUNK, 1, unroll=4)
            def _(r):
                for half in range(2):
                    a = inb[buf, r, pl.ds(half * 32, 16)]
                    c2 = inb[buf, r, pl.ds(half * 32 + 16, 16)]
                    packed = plsc.pack(a, c2,
                                       format=plsc.PackFormat.INTERLEAVED)
                    outb[buf, r, pl.ds(half * 32, 32)] = packed

        in_dma(0, 0).start()
        in_dma(0, 0).wait()

        def pair_body(p, _):
            c = p * 2
            for par in (0, 1):
                cc = c + par

                @pl.when(cc + 1 < n_chunks)
                def _():
                    in_dma(cc + 1, 1 - par).start()

                @pl.when(cc > 1)
                def _():
                    out_dma(cc - 2, par).wait()

                convert(par)
                out_dma(cc, par).start()

                @pl.when(cc + 1 < n_chunks)
                def _():
                    in_dma(cc + 1, 1 - par).wait()
            return 0

        lax.fori_loop(0, n_chunks // 2, pair_body, 0)
        if n_chunks % 2:
            last = n_chunks - 1
            out_dma(last - 2, 0).wait()
            convert(0)
            out_dma(last, 0).start()
            out_dma(last - 1, 1).wait()
            out_dma(last, 0).wait()
        else:
            out_dma(n_chunks - 2, 0).wait()
            out_dma(n_chunks - 1, 1).wait()

    return body(tpad)


def kernel(inputs, table):
    b, k, h, w = inputs.shape
    v = table.shape[0]
    tpad = jnp.pad(table, ((0, 0), (0, 128 - D)))
    tbf = _sc_table_to_bf16(tpad, v)
    idx = inputs.transpose(1, 2, 3, 0).reshape(k, h, w, b // 128, 128)
    out = _sc_bow_embed(idx, tbf, b)
    out = out.transpose(3, 5, 0, 1, 2, 4)
    return out.reshape(b, D, h, w)


# speedup vs baseline: 1.0686x; 1.0686x over previous
import functools

import jax
import jax.numpy as jnp
from jax import lax
from jax.experimental import pallas as pl
from jax.experimental.pallas import tpu as pltpu
from jax.experimental.pallas import tpu_sc as plsc

D = 64
HW = 256
K = 3
BB = 128
OUT_PAD = 129
NW = 32


def _sc_bow_embed(idx, table, batch):
    chunks = batch // BB
    n_units = HW * chunks // NW
    hw_per_w = n_units // chunks

    mesh = plsc.VectorSubcoreMesh(core_axis_name="c", subcore_axis_name="s")

    @functools.partial(
        pl.kernel,
        out_type=jax.ShapeDtypeStruct((D, 16, 2, batch // 128, 8, 128),
                                      jnp.float32),
        mesh=mesh,
        compiler_params=pltpu.CompilerParams(
            needs_layout_passes=False, use_tc_tiling_on_sc=False),
        scratch_types=[
            pltpu.VMEM((4, K, BB), jnp.int32),
            pltpu.VMEM((4, K * BB, D), jnp.float32),
            pltpu.VMEM((D, OUT_PAD), jnp.float32),
            [pltpu.SemaphoreType.DMA] * 4,
            [pltpu.SemaphoreType.DMA] * 4,
            pltpu.SemaphoreType.DMA,
        ],
    )
    def body(idx_hbm, table_hbm, out_hbm, idx_v, rows_v, out_t,
             gsem, isem, osem):
        wid = lax.axis_index("s") * 2 + lax.axis_index("c")
        lane = lax.iota(jnp.int32, 16)
        third = jnp.float32(1.0 / 3.0)
        e_rows = [c * 16 + lane for c in range(4)]

        def unit_hwb(u):
            hw = wid * hw_per_w + (u // chunks)
            cb = u % chunks
            return hw // 16, hw % 16, cb

        def idx_dma(u, buf):
            h, w, cb = unit_hwb(u)
            return pltpu.make_async_copy(
                idx_hbm.at[:, h, w, cb], idx_v.at[buf], isem[buf])

        def fire(buf):
            for k in range(K):
                pltpu.async_copy(
                    table_hbm.at[idx_v.at[buf, k]],
                    rows_v.at[buf, pl.ds(k * BB, BB)],
                    gsem[buf],
                )

        def drain(buf):
            for k in range(K):
                pltpu.make_async_copy(
                    table_hbm.at[idx_v.at[buf, k]],
                    rows_v.at[buf, pl.ds(k * BB, BB)],
                    gsem[buf],
                ).wait()

        def out_dma(u):
            h, w, cb = unit_hwb(u)
            wt, w8 = w // 8, w % 8
            return pltpu.make_async_copy(
                out_t.at[:, pl.ds(0, BB)],
                out_hbm.at[:, h, wt, cb, w8],
                osem,
            )

        def compute(buf):
            rv = rows_v.at[buf]

            @plsc.parallel_loop(0, BB, 1, unroll=4)
            def _(p):
                col = jnp.zeros((16,), jnp.int32) + p
                for c in range(4):
                    sl = pl.ds(c * 16, 16)
                    v = (rv[p, sl] + rv[BB + p, sl]
                         + rv[2 * BB + p, sl]) * third
                    plsc.store_scatter(out_t, [e_rows[c], col], v)

        idx_dma(0, 0).start()
        idx_dma(0, 0).wait()
        fire(0)
        idx_dma(1, 1).start()
        idx_dma(1, 1).wait()
        fire(1)
        idx_dma(2, 2).start()

        def quad_body(qr, _):
            base = qr * 4
            for par in (0, 1, 2, 3):
                u = base + par

                @pl.when(u + 2 < n_units)
                def _():
                    idx_dma(u + 2, (par + 2) & 3).wait()
                    fire((par + 2) & 3)

                @pl.when(u + 3 < n_units)
                def _():
                    idx_dma(u + 3, (par + 3) & 3).start()

                drain(par)

                @pl.when(u > 0)
                def _():
                    out_dma(u - 1).wait()

                compute(par)
                out_dma(u).start()
            return 0

        lax.fori_loop(0, n_units // 4, quad_body, 0)
        out_dma(n_units - 1).wait()

    return body(idx, table)


def kernel(inputs, table):
    b, k, h, w = inputs.shape
    idx = inputs.transpose(1, 2, 3, 0).reshape(k, h, w, b // 128, 128)
    out = _sc_bow_embed(idx, table, b)
    out = out.transpose(3, 5, 0, 1, 2, 4)
    return out.reshape(b, D, h, w)
